# baseline (speedup 1.0000x reference)
import functools

import jax
import jax.numpy as jnp
from jax import lax
from jax.experimental import pallas as pl
from jax.experimental.pallas import tpu as pltpu
from jax.experimental.pallas import tpu_sc as plsc

_N = 10000
_NP = 10240
_K = 20
_G = 128
_E = 320000
_KP = 32
_NC = 2
_NS = 16
_NW = _NC * _NS
_EPT = _E // _NW
_CH = 80
_NCH = _EPT // _CH
_RPT = _NP // _NS


def _dense_body(x_ref, mut_ref, vart_ref, w_ref, s_ref, wcat_ref, bcat_ref,
                src_tab_ref, dst_tab_ref, lp_ref, ll_ref):
    x = x_ref[...]
    ivt = 1.0 / vart_ref[...]
    mut = mut_ref[...]
    a = jnp.dot(x * x, ivt, preferred_element_type=jnp.float32)
    b = jnp.dot(x, mut * ivt, preferred_element_type=jnp.float32)
    c = jnp.sum(mut * mut * ivt, axis=0, keepdims=True)
    s = s_ref[...]
    f = -0.5 * (a - 2.0 * s * b + (s * s) * c)
    w = w_ref[...]
    wmax = jnp.max(w, axis=1, keepdims=True)
    ew = jnp.exp(w - wmax)
    p = ew / jnp.sum(ew, axis=1, keepdims=True)
    ll_ref[...] = (jnp.sum(p * f) / _N)[None, None]
    col = lax.broadcasted_iota(jnp.int32, p.shape, 1)
    pe = jnp.where(col == _K, 1.0, p)
    lp_ref[...] = jnp.where(col < _K, jnp.log(p + 1e-8), 0.0)
    xcat = (jnp.dot(x, wcat_ref[...], preferred_element_type=jnp.float32)
            + bcat_ref[...])
    z8 = jnp.zeros((x.shape[0], 8), jnp.float32)
    src_tab_ref[...] = jnp.concatenate(
        [xcat[:, 0:8], z8, pe, z8, z8], axis=1)
    dst_tab_ref[...] = jnp.concatenate([xcat[:, 8:16], z8, z8, z8], axis=1)


def _edge_body(ei_hbm, stab_hbm, dtab_hbm, att_hbm, zeros_hbm, out_hbm,
               bsh, iball, xs0, xs1, xs2, xs3, xd0, xd1, xd2, xd3,
               pay0, pay1, pay2, pay3, attv, rowb,
               gs0, gs1, gs2, gs3, ss0, ss1, ss2, ss3):
    xs = (xs0, xs1, xs2, xs3)
    xd = (xd0, xd1, xd2, xd3)
    pay = (pay0, pay1, pay2, pay3)
    gs = (gs0, gs1, gs2, gs3)
    ss = (ss0, ss1, ss2, ss3)

    cid = lax.axis_index("c")
    sid = lax.axis_index("s")
    wid = sid * _NC + cid
    rbase = sid * _RPT

    pltpu.sync_copy(zeros_hbm.at[pl.ds(rbase, _RPT)], rowb)
    pltpu.sync_copy(rowb, bsh.at[pl.ds(rbase, _RPT)])
    pltpu.sync_copy(att_hbm, attv)
    plsc.subcore_barrier()
    attvec = attv[...]

    pltpu.sync_copy(ei_hbm.at[:, pl.ds(wid * _NCH, _NCH), :], iball)

    def start_gathers(i, b):
        pltpu.async_copy(stab_hbm.at[iball.at[0, i]], xs[b], gs[b])
        pltpu.async_copy(dtab_hbm.at[iball.at[1, i]], xd[b], gs[b])

    def wait_gathers(i, b):
        pltpu.make_async_copy(stab_hbm.at[iball.at[0, i]], xs[b], gs[b]).wait()
        pltpu.make_async_copy(dtab_hbm.at[iball.at[1, i]], xd[b], gs[b]).wait()

    def wait_scatter(i, b):
        pltpu.make_async_copy(pay[b], bsh.at[iball.at[1, i]], ss[b]).wait()

    def compute_and_scatter(i, b):
        for g in range(_CH // 16):
            rows = lax.iota(jnp.int32, 16) + (g * 16)
            acc = jnp.zeros((16,), jnp.float32)
            for h in range(8):
                ch = jnp.full((16,), h, jnp.int32)
                v = (plsc.load_gather(xs[b], [rows, ch])
                     + plsc.load_gather(xd[b], [rows, ch]))
                z = jnp.maximum(v, 0.2 * v)
                acc = acc + z * attvec[h]
            exg = jnp.exp(acc)
            for l in range(_KP):
                cl = jnp.full((16,), 16 + l, jnp.int32)
                col = plsc.load_gather(xs[b], [rows, cl])
                plsc.store_scatter(pay[b], [rows, jnp.full((16,), l, jnp.int32)],
                                   col * exg)
        pltpu.async_copy(pay[b], bsh.at[iball.at[1, i]], ss[b], add=True)

    for c in range(4):
        start_gathers(c, c)
    for c in range(4):
        wait_gathers(c, c)
        compute_and_scatter(c, c)
        start_gathers(c + 4, c)

    def quad(io, carry):
        i0 = 8 + io * 4
        for b in range(4):
            i = i0 + b - 4
            wait_gathers(i, b)
            wait_scatter(i - 4, b)
            compute_and_scatter(i, b)
            start_gathers(jnp.minimum(i + 4, _NCH - 1), b)
        return carry

    lax.fori_loop(0, (_NCH - 5) // 4, quad, 0)

    wait_gathers(_NCH - 1, 0)
    wait_scatter(_NCH - 5, 0)
    compute_and_scatter(_NCH - 1, 0)
    for b in range(1, 4):
        wait_gathers(_NCH - 1, b)
        wait_scatter(_NCH - 5 + b, b)
    wait_scatter(_NCH - 1, 0)

    plsc.subcore_barrier()
    pltpu.sync_copy(bsh.at[pl.ds(rbase, _RPT)], rowb)
    pltpu.sync_copy(rowb, out_hbm.at[cid, pl.ds(rbase, _RPT)])


def _final_body(bp_ref, lp_ref, ce_ref):
    b = bp_ref[0] + bp_ref[1]
    num = jnp.sum(b * lp_ref[...], axis=1, keepdims=True)
    den = b[:, _K:_K + 1] + 1e-16
    ce_ref[...] = (-jnp.sum(num / den) / _N)[None, None]


@jax.jit
def kernel(X, Mu, Var, edge_index, W, S, W_l, b_l, W_r, b_r, att):
    f32 = jnp.float32
    npad = _NP - _N
    wcat = jnp.concatenate([W_l, W_r], axis=1)
    bcat = jnp.concatenate([b_l, b_r]).reshape(1, 16)
    mu_t = jnp.pad(Mu, ((0, _KP - _K), (0, 0))).T
    var_t = jnp.pad(Var, ((0, _KP - _K), (0, 0)), constant_values=1.0).T
    w32 = jnp.pad(W, ((0, npad), (0, _KP - _K)), constant_values=-1e30)
    xp = jnp.pad(X, ((0, npad), (0, 0)))
    sp = jnp.pad(S, ((0, npad), (0, 0)))
    att16 = jnp.pad(att, (0, 8))

    stab, dtab, lp, ll = pl.pallas_call(
        _dense_body,
        out_shape=(
            jax.ShapeDtypeStruct((_NP, 64), f32),
            jax.ShapeDtypeStruct((_NP, _KP), f32),
            jax.ShapeDtypeStruct((_NP, _KP), f32),
            jax.ShapeDtypeStruct((1, 1), f32),
        ),
    )(xp, mu_t, var_t, w32, sp, wcat, bcat)

    edge_call = functools.partial(
        pl.kernel,
        out_type=jax.ShapeDtypeStruct((_NC, _NP, _KP), f32),
        mesh=plsc.VectorSubcoreMesh(
            core_axis_name="c", subcore_axis_name="s",
            num_cores=_NC, num_subcores=_NS),
        scratch_types=(
            [pltpu.VMEM_SHARED((_NP, _KP), f32),
             pltpu.VMEM((2, _NCH, _CH), jnp.int32)]
            + [pltpu.VMEM((_CH, 64), f32)] * 4
            + [pltpu.VMEM((_CH, _KP), f32)] * 4
            + [pltpu.VMEM((_CH, _KP), f32)] * 4
            + [pltpu.VMEM((16,), f32),
               pltpu.VMEM((_RPT, _KP), f32)]
            + [pltpu.SemaphoreType.DMA] * 8
        ),
        compiler_params=pltpu.CompilerParams(
            needs_layout_passes=False, use_tc_tiling_on_sc=False),
    )(_edge_body)
    ei3 = edge_index.reshape(2, _E // _CH, _CH)
    bparts = edge_call(ei3, stab, dtab, att16, jnp.zeros((_NP, _KP), f32))

    ce = pl.pallas_call(
        _final_body,
        out_shape=jax.ShapeDtypeStruct((1, 1), f32),
    )(bparts, lp)

    return (ll[0, 0], ce[0, 0], stab[:_N, 16:16 + _K])


# speedup vs baseline: 2.6967x; 2.6967x over previous
import functools

import jax
import jax.numpy as jnp
from jax import lax
from jax.experimental import pallas as pl
from jax.experimental.pallas import tpu as pltpu
from jax.experimental.pallas import tpu_sc as plsc

_N = 10000
_NP = 10240
_K = 20
_G = 128
_E = 320000
_KP = 32
_NC = 2
_NS = 16
_NW = _NC * _NS
_EPT = _E // _NW
_CH = 80
_NCH = _EPT // _CH
_RPT = _NP // _NS


def _dense_body(x_ref, mut_ref, vart_ref, w_ref, s_ref, wcat_ref, bcat_ref,
                src_tab_ref, dst_tab_ref, lp_ref, ll_ref):
    x = x_ref[...]
    ivt = 1.0 / vart_ref[...]
    mut = mut_ref[...]
    a = jnp.dot(x * x, ivt, preferred_element_type=jnp.float32)
    b = jnp.dot(x, mut * ivt, preferred_element_type=jnp.float32)
    c = jnp.sum(mut * mut * ivt, axis=0, keepdims=True)
    s = s_ref[...]
    f = -0.5 * (a - 2.0 * s * b + (s * s) * c)
    w = w_ref[...]
    wmax = jnp.max(w, axis=1, keepdims=True)
    ew = jnp.exp(w - wmax)
    p = ew / jnp.sum(ew, axis=1, keepdims=True)
    ll_ref[...] = (jnp.sum(p * f) / _N)[None, None]
    col = lax.broadcasted_iota(jnp.int32, p.shape, 1)
    pe = jnp.where(col == _K, 1.0, p)
    lp_ref[...] = jnp.where(col < _K, jnp.log(p + 1e-8), 0.0)
    xcat = (jnp.dot(x, wcat_ref[...], preferred_element_type=jnp.float32)
            + bcat_ref[...])
    z8 = jnp.zeros((x.shape[0], 8), jnp.float32)
    src_tab_ref[...] = jnp.concatenate(
        [xcat[:, 0:8], z8, pe, z8, z8], axis=1)
    dst_tab_ref[...] = jnp.concatenate([xcat[:, 8:16], z8, z8, z8], axis=1)


def _edge_body(ei_hbm, stab_hbm, dtab_hbm, att_hbm, zeros_hbm, out_hbm,
               bsh, iball, xs0, xs1, xs2, xs3, xd0, xd1, xd2, xd3,
               pay0, pay1, pay2, pay3, attv, rowb,
               gs0, gs1, gs2, gs3, ss0, ss1, ss2, ss3):
    xs = (xs0, xs1, xs2, xs3)
    xd = (xd0, xd1, xd2, xd3)
    pay = (pay0, pay1, pay2, pay3)
    gs = (gs0, gs1, gs2, gs3)
    ss = (ss0, ss1, ss2, ss3)

    cid = lax.axis_index("c")
    sid = lax.axis_index("s")
    wid = sid * _NC + cid
    rbase = sid * _RPT

    pltpu.sync_copy(zeros_hbm.at[pl.ds(rbase, _RPT)], rowb)
    pltpu.sync_copy(rowb, bsh.at[pl.ds(rbase, _RPT)])
    pltpu.sync_copy(att_hbm, attv)
    plsc.subcore_barrier()
    attvec = attv[...]

    pltpu.sync_copy(ei_hbm.at[:, pl.ds(wid * _NCH, _NCH), :], iball)

    def start_gathers(i, b):
        pltpu.async_copy(stab_hbm.at[iball.at[0, i]], xs[b], gs[b])
        pltpu.async_copy(dtab_hbm.at[iball.at[1, i]], xd[b], gs[b])

    def wait_gathers(i, b):
        pltpu.make_async_copy(stab_hbm.at[iball.at[0, i]], xs[b], gs[b]).wait()
        pltpu.make_async_copy(dtab_hbm.at[iball.at[1, i]], xd[b], gs[b]).wait()

    def wait_scatter(i, b):
        pltpu.make_async_copy(pay[b], bsh.at[iball.at[1, i]], ss[b]).wait()

    def compute_and_scatter(i, b):
        for g in range(_CH // 16):
            rows = lax.iota(jnp.int32, 16) + (g * 16)
            acc = jnp.zeros((16,), jnp.float32)
            for h in range(8):
                ch = jnp.full((16,), h, jnp.int32)
                v = (plsc.load_gather(xs[b], [rows, ch])
                     + plsc.load_gather(xd[b], [rows, ch]))
                z = jnp.maximum(v, 0.2 * v)
                acc = acc + z * attvec[h]
            exg = jnp.exp(acc)
            for j in range(16):
                e = g * 16 + j
                ex_e = exg[j]
                pay[b][e, 0:16] = xs[b][e, 16:32] * ex_e
                pay[b][e, 16:32] = xs[b][e, 32:48] * ex_e
        pltpu.async_copy(pay[b], bsh.at[iball.at[1, i]], ss[b], add=True)

    for c in range(4):
        start_gathers(c, c)
    for c in range(4):
        wait_gathers(c, c)
        compute_and_scatter(c, c)
        start_gathers(c + 4, c)

    def quad(io, carry):
        i0 = 8 + io * 4
        for b in range(4):
            i = i0 + b - 4
            wait_gathers(i, b)
            wait_scatter(i - 4, b)
            compute_and_scatter(i, b)
            start_gathers(jnp.minimum(i + 4, _NCH - 1), b)
        return carry

    lax.fori_loop(0, (_NCH - 5) // 4, quad, 0)

    wait_gathers(_NCH - 1, 0)
    wait_scatter(_NCH - 5, 0)
    compute_and_scatter(_NCH - 1, 0)
    for b in range(1, 4):
        wait_gathers(_NCH - 1, b)
        wait_scatter(_NCH - 5 + b, b)
    wait_scatter(_NCH - 1, 0)

    plsc.subcore_barrier()
    pltpu.sync_copy(bsh.at[pl.ds(rbase, _RPT)], rowb)
    pltpu.sync_copy(rowb, out_hbm.at[cid, pl.ds(rbase, _RPT)])


def _final_body(bp_ref, lp_ref, ce_ref):
    b = bp_ref[0] + bp_ref[1]
    num = jnp.sum(b * lp_ref[...], axis=1, keepdims=True)
    den = b[:, _K:_K + 1] + 1e-16
    ce_ref[...] = (-jnp.sum(num / den) / _N)[None, None]


@jax.jit
def kernel(X, Mu, Var, edge_index, W, S, W_l, b_l, W_r, b_r, att):
    f32 = jnp.float32
    npad = _NP - _N
    wcat = jnp.concatenate([W_l, W_r], axis=1)
    bcat = jnp.concatenate([b_l, b_r]).reshape(1, 16)
    mu_t = jnp.pad(Mu, ((0, _KP - _K), (0, 0))).T
    var_t = jnp.pad(Var, ((0, _KP - _K), (0, 0)), constant_values=1.0).T
    w32 = jnp.pad(W, ((0, npad), (0, _KP - _K)), constant_values=-1e30)
    xp = jnp.pad(X, ((0, npad), (0, 0)))
    sp = jnp.pad(S, ((0, npad), (0, 0)))
    att16 = jnp.pad(att, (0, 8))

    stab, dtab, lp, ll = pl.pallas_call(
        _dense_body,
        out_shape=(
            jax.ShapeDtypeStruct((_NP, 64), f32),
            jax.ShapeDtypeStruct((_NP, _KP), f32),
            jax.ShapeDtypeStruct((_NP, _KP), f32),
            jax.ShapeDtypeStruct((1, 1), f32),
        ),
    )(xp, mu_t, var_t, w32, sp, wcat, bcat)

    edge_call = functools.partial(
        pl.kernel,
        out_type=jax.ShapeDtypeStruct((_NC, _NP, _KP), f32),
        mesh=plsc.VectorSubcoreMesh(
            core_axis_name="c", subcore_axis_name="s",
            num_cores=_NC, num_subcores=_NS),
        scratch_types=(
            [pltpu.VMEM_SHARED((_NP, _KP), f32),
             pltpu.VMEM((2, _NCH, _CH), jnp.int32)]
            + [pltpu.VMEM((_CH, 64), f32)] * 4
            + [pltpu.VMEM((_CH, _KP), f32)] * 4
            + [pltpu.VMEM((_CH, _KP), f32)] * 4
            + [pltpu.VMEM((16,), f32),
               pltpu.VMEM((_RPT, _KP), f32)]
            + [pltpu.SemaphoreType.DMA] * 8
        ),
        compiler_params=pltpu.CompilerParams(
            needs_layout_passes=False, use_tc_tiling_on_sc=False),
    )(_edge_body)
    ei3 = edge_index.reshape(2, _E // _CH, _CH)
    bparts = edge_call(ei3, stab, dtab, att16, jnp.zeros((_NP, _KP), f32))

    ce = pl.pallas_call(
        _final_body,
        out_shape=jax.ShapeDtypeStruct((1, 1), f32),
    )(bparts, lp)

    return (ll[0, 0], ce[0, 0], stab[:_N, 16:16 + _K])


# speedup vs baseline: 3.1493x; 1.1678x over previous
import functools

import jax
import jax.numpy as jnp
from jax import lax
from jax.experimental import pallas as pl
from jax.experimental.pallas import tpu as pltpu
from jax.experimental.pallas import tpu_sc as plsc

_N = 10000
_NP = 10240
_K = 20
_G = 128
_E = 320000
_KP = 32
_NC = 2
_NS = 16
_NW = _NC * _NS
_EPT = _E // _NW
_CH = 80
_NCH = _EPT // _CH
_RPT = _NP // _NS


def _dense_body(x_ref, mut_ref, vart_ref, w_ref, s_ref, wcat_ref, bcat_ref,
                src_tab_ref, dst_tab_ref, lp_ref, ll_ref):
    x = x_ref[...]
    ivt = 1.0 / vart_ref[...]
    mut = mut_ref[...]
    a = jnp.dot(x * x, ivt, preferred_element_type=jnp.float32)
    b = jnp.dot(x, mut * ivt, preferred_element_type=jnp.float32)
    c = jnp.sum(mut * mut * ivt, axis=0, keepdims=True)
    s = s_ref[...]
    f = -0.5 * (a - 2.0 * s * b + (s * s) * c)
    w = w_ref[...]
    wmax = jnp.max(w, axis=1, keepdims=True)
    ew = jnp.exp(w - wmax)
    p = ew / jnp.sum(ew, axis=1, keepdims=True)
    ll_ref[...] = (jnp.sum(p * f) / _N)[None, None]
    col = lax.broadcasted_iota(jnp.int32, p.shape, 1)
    pe = jnp.where(col == _K, 1.0, p)
    lp_ref[...] = jnp.where(col < _K, jnp.log(p + 1e-8), 0.0)
    xcat = (jnp.dot(x, wcat_ref[...], preferred_element_type=jnp.float32)
            + bcat_ref[...])
    z8 = jnp.zeros((x.shape[0], 8), jnp.float32)
    src_tab_ref[...] = jnp.concatenate([xcat[:, 0:8], z8, pe], axis=1)
    dst_tab_ref[...] = jnp.concatenate([xcat[:, 8:16], z8], axis=1)


def _edge_body(ei_hbm, stab_hbm, dtab_hbm, att_hbm, zeros_hbm, out_hbm,
               bsh, iball, xs0, xs1, xs2, xs3, xd0, xd1, xd2, xd3,
               pay0, pay1, pay2, pay3, attv, rowb,
               gs0, gs1, gs2, gs3, ss0, ss1, ss2, ss3):
    xs = (xs0, xs1, xs2, xs3)
    xd = (xd0, xd1, xd2, xd3)
    pay = (pay0, pay1, pay2, pay3)
    gs = (gs0, gs1, gs2, gs3)
    ss = (ss0, ss1, ss2, ss3)

    cid = lax.axis_index("c")
    sid = lax.axis_index("s")
    wid = sid * _NC + cid
    rbase = sid * _RPT

    pltpu.sync_copy(zeros_hbm.at[pl.ds(rbase, _RPT)], rowb)
    pltpu.sync_copy(rowb, bsh.at[pl.ds(rbase, _RPT)])
    pltpu.sync_copy(att_hbm, attv)
    plsc.subcore_barrier()
    attvec = attv[...]

    pltpu.sync_copy(ei_hbm.at[:, pl.ds(wid * _NCH, _NCH), :], iball)

    def start_gathers(i, b):
        pltpu.async_copy(stab_hbm.at[iball.at[0, i]], xs[b], gs[b])
        pltpu.async_copy(dtab_hbm.at[iball.at[1, i]], xd[b], gs[b])

    def wait_gathers(i, b):
        pltpu.make_async_copy(stab_hbm.at[iball.at[0, i]], xs[b], gs[b]).wait()
        pltpu.make_async_copy(dtab_hbm.at[iball.at[1, i]], xd[b], gs[b]).wait()

    def wait_scatter(i, b):
        pltpu.make_async_copy(pay[b], bsh.at[iball.at[1, i]], ss[b]).wait()

    def compute_and_scatter(i, b):
        for g in range(_CH // 16):
            rows = lax.iota(jnp.int32, 16) + (g * 16)
            acc = jnp.zeros((16,), jnp.float32)
            for h in range(8):
                ch = jnp.full((16,), h, jnp.int32)
                v = (plsc.load_gather(xs[b], [rows, ch])
                     + plsc.load_gather(xd[b], [rows, ch]))
                z = jnp.maximum(v, 0.2 * v)
                acc = acc + z * attvec[h]
            exg = jnp.exp(acc)
            for j in range(16):
                e = g * 16 + j
                ex_e = exg[j]
                pay[b][e, 0:16] = xs[b][e, 16:32] * ex_e
                pay[b][e, 16:32] = xs[b][e, 32:48] * ex_e
        pltpu.async_copy(pay[b], bsh.at[iball.at[1, i]], ss[b], add=True)

    for c in range(4):
        start_gathers(c, c)
    for c in range(4):
        wait_gathers(c, c)
        compute_and_scatter(c, c)
        start_gathers(c + 4, c)

    def quad(io, carry):
        i0 = 8 + io * 4
        for b in range(4):
            i = i0 + b - 4
            wait_gathers(i, b)
            wait_scatter(i - 4, b)
            compute_and_scatter(i, b)
            start_gathers(jnp.minimum(i + 4, _NCH - 1), b)
        return carry

    lax.fori_loop(0, (_NCH - 5) // 4, quad, 0)

    wait_gathers(_NCH - 1, 0)
    wait_scatter(_NCH - 5, 0)
    compute_and_scatter(_NCH - 1, 0)
    for b in range(1, 4):
        wait_gathers(_NCH - 1, b)
        wait_scatter(_NCH - 5 + b, b)
    wait_scatter(_NCH - 1, 0)

    plsc.subcore_barrier()
    pltpu.sync_copy(bsh.at[pl.ds(rbase, _RPT)], rowb)
    pltpu.sync_copy(rowb, out_hbm.at[cid, pl.ds(rbase, _RPT)])


def _final_body(bp_ref, lp_ref, ce_ref):
    b = bp_ref[0] + bp_ref[1]
    num = jnp.sum(b * lp_ref[...], axis=1, keepdims=True)
    den = b[:, _K:_K + 1] + 1e-16
    ce_ref[...] = (-jnp.sum(num / den) / _N)[None, None]


@jax.jit
def kernel(X, Mu, Var, edge_index, W, S, W_l, b_l, W_r, b_r, att):
    f32 = jnp.float32
    npad = _NP - _N
    wcat = jnp.concatenate([W_l, W_r], axis=1)
    bcat = jnp.concatenate([b_l, b_r]).reshape(1, 16)
    mu_t = jnp.pad(Mu, ((0, _KP - _K), (0, 0))).T
    var_t = jnp.pad(Var, ((0, _KP - _K), (0, 0)), constant_values=1.0).T
    w32 = jnp.pad(W, ((0, npad), (0, _KP - _K)), constant_values=-1e30)
    xp = jnp.pad(X, ((0, npad), (0, 0)))
    sp = jnp.pad(S, ((0, npad), (0, 0)))
    att16 = jnp.pad(att, (0, 8))

    stab, dtab, lp, ll = pl.pallas_call(
        _dense_body,
        out_shape=(
            jax.ShapeDtypeStruct((_NP, 48), f32),
            jax.ShapeDtypeStruct((_NP, 16), f32),
            jax.ShapeDtypeStruct((_NP, _KP), f32),
            jax.ShapeDtypeStruct((1, 1), f32),
        ),
    )(xp, mu_t, var_t, w32, sp, wcat, bcat)

    edge_call = functools.partial(
        pl.kernel,
        out_type=jax.ShapeDtypeStruct((_NC, _NP, _KP), f32),
        mesh=plsc.VectorSubcoreMesh(
            core_axis_name="c", subcore_axis_name="s",
            num_cores=_NC, num_subcores=_NS),
        scratch_types=(
            [pltpu.VMEM_SHARED((_NP, _KP), f32),
             pltpu.VMEM((2, _NCH, _CH), jnp.int32)]
            + [pltpu.VMEM((_CH, 48), f32)] * 4
            + [pltpu.VMEM((_CH, 16), f32)] * 4
            + [pltpu.VMEM((_CH, _KP), f32)] * 4
            + [pltpu.VMEM((16,), f32),
               pltpu.VMEM((_RPT, _KP), f32)]
            + [pltpu.SemaphoreType.DMA] * 8
        ),
        compiler_params=pltpu.CompilerParams(
            needs_layout_passes=False, use_tc_tiling_on_sc=False),
    )(_edge_body)
    ei3 = edge_index.reshape(2, _E // _CH, _CH)
    bparts = edge_call(ei3, stab, dtab, att16, jnp.zeros((_NP, _KP), f32))

    ce = pl.pallas_call(
        _final_body,
        out_shape=jax.ShapeDtypeStruct((1, 1), f32),
    )(bparts, lp)

    return (ll[0, 0], ce[0, 0], stab[:_N, 16:16 + _K])


# speedup vs baseline: 3.2828x; 1.0424x over previous
import functools

import jax
import jax.numpy as jnp
from jax import lax
from jax.experimental import pallas as pl
from jax.experimental.pallas import tpu as pltpu
from jax.experimental.pallas import tpu_sc as plsc

_N = 10000
_NP = 10240
_K = 20
_G = 128
_E = 320000
_KP = 32
_NC = 2
_NS = 16
_NW = _NC * _NS
_EPT = _E // _NW
_CH = 80
_NCH = _EPT // _CH
_RPT = _NP // _NS


def _dense_body(x_ref, mut_ref, vart_ref, w_ref, s_ref, wcat_ref, bcat_ref,
                src_tab_ref, dst_tab_ref, lp_ref, ll_ref):
    x = x_ref[...]
    ivt = 1.0 / vart_ref[...]
    mut = mut_ref[...]
    a = jnp.dot(x * x, ivt, preferred_element_type=jnp.float32)
    b = jnp.dot(x, mut * ivt, preferred_element_type=jnp.float32)
    c = jnp.sum(mut * mut * ivt, axis=0, keepdims=True)
    s = s_ref[...]
    f = -0.5 * (a - 2.0 * s * b + (s * s) * c)
    w = w_ref[...]
    wmax = jnp.max(w, axis=1, keepdims=True)
    ew = jnp.exp(w - wmax)
    p = ew / jnp.sum(ew, axis=1, keepdims=True)
    ll_ref[...] = (jnp.sum(p * f) / _N)[None, None]
    col = lax.broadcasted_iota(jnp.int32, p.shape, 1)
    pe = jnp.where(col == _K, 1.0, p)
    lp_ref[...] = jnp.where(col < _K, jnp.log(p + 1e-8), 0.0)
    xcat = (jnp.dot(x, wcat_ref[...], preferred_element_type=jnp.float32)
            + bcat_ref[...])
    src_tab_ref[...] = jnp.concatenate([xcat[:, 0:8], pe], axis=1)
    dst_tab_ref[...] = xcat[:, 8:16]


def _edge_body(ei_hbm, stab_hbm, dtab_hbm, att_hbm, zeros_hbm, out_hbm,
               bsh, iball, xs0, xs1, xs2, xs3, xd0, xd1, xd2, xd3,
               pay0, pay1, pay2, pay3, attv, rowb,
               gs0, gs1, gs2, gs3, ss0, ss1, ss2, ss3):
    xs = (xs0, xs1, xs2, xs3)
    xd = (xd0, xd1, xd2, xd3)
    pay = (pay0, pay1, pay2, pay3)
    gs = (gs0, gs1, gs2, gs3)
    ss = (ss0, ss1, ss2, ss3)

    cid = lax.axis_index("c")
    sid = lax.axis_index("s")
    wid = sid * _NC + cid
    rbase = sid * _RPT

    pltpu.sync_copy(zeros_hbm.at[pl.ds(rbase, _RPT)], rowb)
    pltpu.sync_copy(rowb, bsh.at[pl.ds(rbase, _RPT)])
    pltpu.sync_copy(att_hbm, attv)
    plsc.subcore_barrier()
    attvec = attv[...]

    pltpu.sync_copy(ei_hbm.at[:, pl.ds(wid * _NCH, _NCH), :], iball)

    def start_gathers(i, b):
        pltpu.async_copy(stab_hbm.at[iball.at[0, i]], xs[b], gs[b])
        pltpu.async_copy(dtab_hbm.at[iball.at[1, i]], xd[b], gs[b])

    def wait_gathers(i, b):
        pltpu.make_async_copy(stab_hbm.at[iball.at[0, i]], xs[b], gs[b]).wait()
        pltpu.make_async_copy(dtab_hbm.at[iball.at[1, i]], xd[b], gs[b]).wait()

    def wait_scatter(i, b):
        pltpu.make_async_copy(pay[b], bsh.at[iball.at[1, i]], ss[b]).wait()

    def compute_and_scatter(i, b):
        for g in range(_CH // 16):
            rows = lax.iota(jnp.int32, 16) + (g * 16)
            acc = jnp.zeros((16,), jnp.float32)
            for h in range(8):
                ch = jnp.full((16,), h, jnp.int32)
                v = (plsc.load_gather(xs[b], [rows, ch])
                     + plsc.load_gather(xd[b], [rows, ch]))
                z = jnp.maximum(v, 0.2 * v)
                acc = acc + z * attvec[h]
            exg = jnp.exp(acc)
            for j in range(16):
                e = g * 16 + j
                ex_e = exg[j]
                pay[b][e, 0:16] = xs[b][e, 8:24] * ex_e
                pay[b][e, 16:32] = xs[b][e, 24:40] * ex_e
        pltpu.async_copy(pay[b], bsh.at[iball.at[1, i]], ss[b], add=True)

    for c in range(4):
        start_gathers(c, c)
    for c in range(4):
        wait_gathers(c, c)
        compute_and_scatter(c, c)
        start_gathers(c + 4, c)

    def quad(io, carry):
        i0 = 8 + io * 4
        for b in range(4):
            i = i0 + b - 4
            wait_gathers(i, b)
            wait_scatter(i - 4, b)
            compute_and_scatter(i, b)
            start_gathers(jnp.minimum(i + 4, _NCH - 1), b)
        return carry

    lax.fori_loop(0, (_NCH - 5) // 4, quad, 0)

    wait_gathers(_NCH - 1, 0)
    wait_scatter(_NCH - 5, 0)
    compute_and_scatter(_NCH - 1, 0)
    for b in range(1, 4):
        wait_gathers(_NCH - 1, b)
        wait_scatter(_NCH - 5 + b, b)
    wait_scatter(_NCH - 1, 0)

    plsc.subcore_barrier()
    pltpu.sync_copy(bsh.at[pl.ds(rbase, _RPT)], rowb)
    pltpu.sync_copy(rowb, out_hbm.at[cid, pl.ds(rbase, _RPT)])


def _final_body(bp_ref, lp_ref, ce_ref):
    b = bp_ref[0] + bp_ref[1]
    num = jnp.sum(b * lp_ref[...], axis=1, keepdims=True)
    den = b[:, _K:_K + 1] + 1e-16
    ce_ref[...] = (-jnp.sum(num / den) / _N)[None, None]


@jax.jit
def kernel(X, Mu, Var, edge_index, W, S, W_l, b_l, W_r, b_r, att):
    f32 = jnp.float32
    npad = _NP - _N
    wcat = jnp.concatenate([W_l, W_r], axis=1)
    bcat = jnp.concatenate([b_l, b_r]).reshape(1, 16)
    mu_t = jnp.pad(Mu, ((0, _KP - _K), (0, 0))).T
    var_t = jnp.pad(Var, ((0, _KP - _K), (0, 0)), constant_values=1.0).T
    w32 = jnp.pad(W, ((0, npad), (0, _KP - _K)), constant_values=-1e30)
    xp = jnp.pad(X, ((0, npad), (0, 0)))
    sp = jnp.pad(S, ((0, npad), (0, 0)))
    att16 = jnp.pad(att, (0, 8))

    stab, dtab, lp, ll = pl.pallas_call(
        _dense_body,
        out_shape=(
            jax.ShapeDtypeStruct((_NP, 40), f32),
            jax.ShapeDtypeStruct((_NP, 8), f32),
            jax.ShapeDtypeStruct((_NP, _KP), f32),
            jax.ShapeDtypeStruct((1, 1), f32),
        ),
    )(xp, mu_t, var_t, w32, sp, wcat, bcat)

    edge_call = functools.partial(
        pl.kernel,
        out_type=jax.ShapeDtypeStruct((_NC, _NP, _KP), f32),
        mesh=plsc.VectorSubcoreMesh(
            core_axis_name="c", subcore_axis_name="s",
            num_cores=_NC, num_subcores=_NS),
        scratch_types=(
            [pltpu.VMEM_SHARED((_NP, _KP), f32),
             pltpu.VMEM((2, _NCH, _CH), jnp.int32)]
            + [pltpu.VMEM((_CH, 40), f32)] * 4
            + [pltpu.VMEM((_CH, 8), f32)] * 4
            + [pltpu.VMEM((_CH, _KP), f32)] * 4
            + [pltpu.VMEM((16,), f32),
               pltpu.VMEM((_RPT, _KP), f32)]
            + [pltpu.SemaphoreType.DMA] * 8
        ),
        compiler_params=pltpu.CompilerParams(
            needs_layout_passes=False, use_tc_tiling_on_sc=False),
    )(_edge_body)
    ei3 = edge_index.reshape(2, _E // _CH, _CH)
    bparts = edge_call(ei3, stab, dtab, att16, jnp.zeros((_NP, _KP), f32))

    ce = pl.pallas_call(
        _final_body,
        out_shape=jax.ShapeDtypeStruct((1, 1), f32),
    )(bparts, lp)

    return (ll[0, 0], ce[0, 0], stab[:_N, 8:8 + _K])


# speedup vs baseline: 3.2911x; 1.0025x over previous
import functools

import jax
import jax.numpy as jnp
from jax import lax
from jax.experimental import pallas as pl
from jax.experimental.pallas import tpu as pltpu
from jax.experimental.pallas import tpu_sc as plsc

_N = 10000
_NP = 10240
_K = 20
_G = 128
_E = 320000
_KP = 32
_NC = 2
_NS = 16
_NW = _NC * _NS
_EPT = _E // _NW
_CH = 80
_NCH = _EPT // _CH
_RPT = _NP // _NS


def _dense_body(x_ref, mut_ref, vart_ref, w_ref, s_ref, wcat_ref, bcat_ref,
                src_tab_ref, dst_tab_ref, lp_ref, pe_ref, ll_ref):
    x = x_ref[...]
    ivt = 1.0 / vart_ref[...]
    mut = mut_ref[...]
    a = jnp.dot(x * x, ivt, preferred_element_type=jnp.float32)
    b = jnp.dot(x, mut * ivt, preferred_element_type=jnp.float32)
    c = jnp.sum(mut * mut * ivt, axis=0, keepdims=True)
    s = s_ref[...]
    f = -0.5 * (a - 2.0 * s * b + (s * s) * c)
    w = w_ref[...]
    wmax = jnp.max(w, axis=1, keepdims=True)
    ew = jnp.exp(w - wmax)
    p = ew / jnp.sum(ew, axis=1, keepdims=True)
    ll_ref[...] = (jnp.sum(p * f) / _N)[None, None]
    col = lax.broadcasted_iota(jnp.int32, p.shape, 1)
    pe = jnp.where(col == _K, 1.0, p)
    pe_ref[...] = pe
    lp_ref[...] = jnp.where(col < _K, jnp.log(p + 1e-8), 0.0)
    xcat = (jnp.dot(x, wcat_ref[...], preferred_element_type=jnp.float32)
            + bcat_ref[...])
    bl = lax.bitcast_convert_type(pe[:, 0:16], jnp.int32)
    bh = lax.bitcast_convert_type(pe[:, 16:32], jnp.int32)
    rl = (bl + 0x7FFF + ((bl >> 16) & 1)) >> 16
    rh = (bh + 0x7FFF + ((bh >> 16) & 1)) >> 16
    packed = (rh << 16) | (rl & 0xFFFF)
    pe_packed = lax.bitcast_convert_type(packed, jnp.float32)
    src_tab_ref[...] = jnp.concatenate([xcat[:, 0:8], pe_packed], axis=1)
    dst_tab_ref[...] = xcat[:, 8:16]


def _edge_body(ei_hbm, stab_hbm, dtab_hbm, att_hbm, zeros_hbm, out_hbm,
               bsh, iball, xs0, xs1, xs2, xs3, xd0, xd1, xd2, xd3,
               pay0, pay1, pay2, pay3, attv, rowb,
               gs0, gs1, gs2, gs3, ss0, ss1, ss2, ss3):
    xs = (xs0, xs1, xs2, xs3)
    xd = (xd0, xd1, xd2, xd3)
    pay = (pay0, pay1, pay2, pay3)
    gs = (gs0, gs1, gs2, gs3)
    ss = (ss0, ss1, ss2, ss3)

    cid = lax.axis_index("c")
    sid = lax.axis_index("s")
    wid = sid * _NC + cid
    rbase = sid * _RPT

    pltpu.sync_copy(zeros_hbm.at[pl.ds(rbase, _RPT)], rowb)
    pltpu.sync_copy(rowb, bsh.at[pl.ds(rbase, _RPT)])
    pltpu.sync_copy(att_hbm, attv)
    plsc.subcore_barrier()
    attvec = attv[...]

    pltpu.sync_copy(ei_hbm.at[:, pl.ds(wid * _NCH, _NCH), :], iball)

    def start_gathers(i, b):
        pltpu.async_copy(stab_hbm.at[iball.at[0, i]], xs[b], gs[b])
        pltpu.async_copy(dtab_hbm.at[iball.at[1, i]], xd[b], gs[b])

    def wait_gathers(i, b):
        pltpu.make_async_copy(stab_hbm.at[iball.at[0, i]], xs[b], gs[b]).wait()
        pltpu.make_async_copy(dtab_hbm.at[iball.at[1, i]], xd[b], gs[b]).wait()

    def wait_scatter(i, b):
        pltpu.make_async_copy(pay[b], bsh.at[iball.at[1, i]], ss[b]).wait()

    def compute_and_scatter(i, b):
        for g in range(_CH // 16):
            rows = lax.iota(jnp.int32, 16) + (g * 16)
            acc = jnp.zeros((16,), jnp.float32)
            for h in range(8):
                ch = jnp.full((16,), h, jnp.int32)
                v = (plsc.load_gather(xs[b], [rows, ch])
                     + plsc.load_gather(xd[b], [rows, ch]))
                z = jnp.maximum(v, 0.2 * v)
                acc = acc + z * attvec[h]
            exg = jnp.exp(acc)
            for j in range(16):
                e = g * 16 + j
                ex_e = exg[j]
                w = plsc.bitcast(xs[b][e, 8:24], jnp.int32)
                lo = plsc.bitcast(w << 16, jnp.float32)
                hi = plsc.bitcast(w & jnp.int32(-65536), jnp.float32)
                pay[b][e, 0:16] = lo * ex_e
                pay[b][e, 16:32] = hi * ex_e
        pltpu.async_copy(pay[b], bsh.at[iball.at[1, i]], ss[b], add=True)

    for c in range(4):
        start_gathers(c, c)
    for c in range(4):
        wait_gathers(c, c)
        compute_and_scatter(c, c)
        start_gathers(c + 4, c)

    def quad(io, carry):
        i0 = 8 + io * 4
        for b in range(4):
            i = i0 + b - 4
            wait_gathers(i, b)
            wait_scatter(i - 4, b)
            compute_and_scatter(i, b)
            start_gathers(jnp.minimum(i + 4, _NCH - 1), b)
        return carry

    lax.fori_loop(0, (_NCH - 5) // 4, quad, 0)

    wait_gathers(_NCH - 1, 0)
    wait_scatter(_NCH - 5, 0)
    compute_and_scatter(_NCH - 1, 0)
    for b in range(1, 4):
        wait_gathers(_NCH - 1, b)
        wait_scatter(_NCH - 5 + b, b)
    wait_scatter(_NCH - 1, 0)

    plsc.subcore_barrier()
    pltpu.sync_copy(bsh.at[pl.ds(rbase, _RPT)], rowb)
    pltpu.sync_copy(rowb, out_hbm.at[cid, pl.ds(rbase, _RPT)])


def _final_body(bp_ref, lp_ref, ce_ref):
    b = bp_ref[0] + bp_ref[1]
    num = jnp.sum(b * lp_ref[...], axis=1, keepdims=True)
    den = b[:, _K:_K + 1] + 1e-16
    ce_ref[...] = (-jnp.sum(num / den) / _N)[None, None]


@jax.jit
def kernel(X, Mu, Var, edge_index, W, S, W_l, b_l, W_r, b_r, att):
    f32 = jnp.float32
    npad = _NP - _N
    wcat = jnp.concatenate([W_l, W_r], axis=1)
    bcat = jnp.concatenate([b_l, b_r]).reshape(1, 16)
    mu_t = jnp.pad(Mu, ((0, _KP - _K), (0, 0))).T
    var_t = jnp.pad(Var, ((0, _KP - _K), (0, 0)), constant_values=1.0).T
    w32 = jnp.pad(W, ((0, npad), (0, _KP - _K)), constant_values=-1e30)
    xp = jnp.pad(X, ((0, npad), (0, 0)))
    sp = jnp.pad(S, ((0, npad), (0, 0)))
    att16 = jnp.pad(att, (0, 8))

    stab, dtab, lp, pe, ll = pl.pallas_call(
        _dense_body,
        out_shape=(
            jax.ShapeDtypeStruct((_NP, 24), f32),
            jax.ShapeDtypeStruct((_NP, 8), f32),
            jax.ShapeDtypeStruct((_NP, _KP), f32),
            jax.ShapeDtypeStruct((_NP, _KP), f32),
            jax.ShapeDtypeStruct((1, 1), f32),
        ),
    )(xp, mu_t, var_t, w32, sp, wcat, bcat)

    edge_call = functools.partial(
        pl.kernel,
        out_type=jax.ShapeDtypeStruct((_NC, _NP, _KP), f32),
        mesh=plsc.VectorSubcoreMesh(
            core_axis_name="c", subcore_axis_name="s",
            num_cores=_NC, num_subcores=_NS),
        scratch_types=(
            [pltpu.VMEM_SHARED((_NP, _KP), f32),
             pltpu.VMEM((2, _NCH, _CH), jnp.int32)]
            + [pltpu.VMEM((_CH, 24), f32)] * 4
            + [pltpu.VMEM((_CH, 8), f32)] * 4
            + [pltpu.VMEM((_CH, _KP), f32)] * 4
            + [pltpu.VMEM((16,), f32),
               pltpu.VMEM((_RPT, _KP), f32)]
            + [pltpu.SemaphoreType.DMA] * 8
        ),
        compiler_params=pltpu.CompilerParams(
            needs_layout_passes=False, use_tc_tiling_on_sc=False),
    )(_edge_body)
    ei3 = edge_index.reshape(2, _E // _CH, _CH)
    bparts = edge_call(ei3, stab, dtab, att16, jnp.zeros((_NP, _KP), f32))

    ce = pl.pallas_call(
        _final_body,
        out_shape=jax.ShapeDtypeStruct((1, 1), f32),
    )(bparts, lp)

    return (ll[0, 0], ce[0, 0], pe[:_N, :_K])


# speedup vs baseline: 3.3469x; 1.0170x over previous
import functools

import jax
import jax.numpy as jnp
from jax import lax
from jax.experimental import pallas as pl
from jax.experimental.pallas import tpu as pltpu
from jax.experimental.pallas import tpu_sc as plsc

_N = 10000
_NP = 10240
_K = 20
_G = 128
_E = 320000
_KP = 32
_NC = 2
_NS = 16
_NW = _NC * _NS
_EPT = _E // _NW
_CH = 80
_NCH = _EPT // _CH
_RPT = _NP // _NS


def _dense_body(x_ref, mut_ref, vart_ref, w_ref, s_ref, wcat_ref, bcat_ref,
                src_tab_ref, dst_tab_ref, lp_ref, pe_ref, ll_ref):
    x = x_ref[...]
    ivt = 1.0 / vart_ref[...]
    mut = mut_ref[...]
    a = jnp.dot(x * x, ivt, preferred_element_type=jnp.float32)
    b = jnp.dot(x, mut * ivt, preferred_element_type=jnp.float32)
    c = jnp.sum(mut * mut * ivt, axis=0, keepdims=True)
    s = s_ref[...]
    f = -0.5 * (a - 2.0 * s * b + (s * s) * c)
    w = w_ref[...]
    wmax = jnp.max(w, axis=1, keepdims=True)
    ew = jnp.exp(w - wmax)
    p = ew / jnp.sum(ew, axis=1, keepdims=True)
    ll_ref[...] = (jnp.sum(p * f) / _N)[None, None]
    col = lax.broadcasted_iota(jnp.int32, p.shape, 1)
    pe = jnp.where(col == _K, 1.0, p)
    pe_ref[...] = pe
    lp_ref[...] = jnp.where(col < _K, jnp.log(p + 1e-8), 0.0)
    xcat = (jnp.dot(x, wcat_ref[...], preferred_element_type=jnp.float32)
            + bcat_ref[...])
    src_tab_ref[...] = jnp.concatenate([xcat[:, 0:8], pe], axis=1)
    dst_tab_ref[...] = xcat[:, 8:16]


def _edge_body(ei_hbm, stab_hbm, dtab_hbm, att_hbm, zeros_hbm, out_hbm,
               bsh, iball, xs0, xs1, xs2, xs3, xd0, xd1, xd2, xd3,
               pay0, pay1, pay2, pay3, attv, rowb,
               gs0, gs1, gs2, gs3, ss0, ss1, ss2, ss3):
    xs = (xs0, xs1, xs2, xs3)
    xd = (xd0, xd1, xd2, xd3)
    pay = (pay0, pay1, pay2, pay3)
    gs = (gs0, gs1, gs2, gs3)
    ss = (ss0, ss1, ss2, ss3)

    cid = lax.axis_index("c")
    sid = lax.axis_index("s")
    wid = sid * _NC + cid
    rbase = sid * _RPT

    pltpu.sync_copy(zeros_hbm.at[pl.ds(rbase, _RPT)], rowb)
    pltpu.sync_copy(rowb, bsh.at[pl.ds(rbase, _RPT)])
    pltpu.sync_copy(att_hbm, attv)
    plsc.subcore_barrier()
    attvec = attv[...]

    pltpu.sync_copy(ei_hbm.at[:, pl.ds(wid * _NCH, _NCH), :], iball)

    def start_gathers(i, b):
        pltpu.async_copy(stab_hbm.at[iball.at[0, i]], xs[b], gs[b])
        pltpu.async_copy(dtab_hbm.at[iball.at[1, i]], xd[b], gs[b])

    def wait_gathers(i, b):
        pltpu.make_async_copy(stab_hbm.at[iball.at[0, i]], xs[b], gs[b]).wait()
        pltpu.make_async_copy(dtab_hbm.at[iball.at[1, i]], xd[b], gs[b]).wait()

    def wait_scatter(i, b):
        pltpu.make_async_copy(pay[b], bsh.at[iball.at[1, i]], ss[b]).wait()

    def compute_and_scatter(i, b):
        for g in range(_CH // 16):
            rows = lax.iota(jnp.int32, 16) + (g * 16)
            acc = jnp.zeros((16,), jnp.float32)
            for h in range(8):
                ch = jnp.full((16,), h, jnp.int32)
                v = (plsc.load_gather(xs[b], [rows, ch])
                     + plsc.load_gather(xd[b], [rows, ch]))
                z = jnp.maximum(v, 0.2 * v)
                acc = acc + z * attvec[h]
            exg = jnp.exp(acc)
            for j in range(16):
                e = g * 16 + j
                ex_e = exg[j]
                pay[b][e, 0:16] = xs[b][e, 8:24] * ex_e
                pay[b][e, 16:32] = xs[b][e, 24:40] * ex_e
        pltpu.async_copy(pay[b], bsh.at[iball.at[1, i]], ss[b], add=True)

    for c in range(4):
        start_gathers(c, c)
    for c in range(4):
        wait_gathers(c, c)
        compute_and_scatter(c, c)
        start_gathers(c + 4, c)

    def quad(io, carry):
        i0 = 8 + io * 4
        for b in range(4):
            i = i0 + b - 4
            wait_gathers(i, b)
            wait_scatter(i - 4, b)
            compute_and_scatter(i, b)
            start_gathers(jnp.minimum(i + 4, _NCH - 1), b)
        return carry

    lax.fori_loop(0, (_NCH - 5) // 4, quad, 0)

    wait_gathers(_NCH - 1, 0)
    wait_scatter(_NCH - 5, 0)
    compute_and_scatter(_NCH - 1, 0)
    for b in range(1, 4):
        wait_gathers(_NCH - 1, b)
        wait_scatter(_NCH - 5 + b, b)
    wait_scatter(_NCH - 1, 0)

    plsc.subcore_barrier()
    pltpu.sync_copy(bsh.at[pl.ds(rbase, _RPT)], rowb)
    pltpu.sync_copy(rowb, out_hbm.at[cid, pl.ds(rbase, _RPT)])


def _final_body(bp_ref, lp_ref, ce_ref):
    b = bp_ref[0] + bp_ref[1]
    num = jnp.sum(b * lp_ref[...], axis=1, keepdims=True)
    den = b[:, _K:_K + 1] + 1e-16
    ce_ref[...] = (-jnp.sum(num / den) / _N)[None, None]


@jax.jit
def kernel(X, Mu, Var, edge_index, W, S, W_l, b_l, W_r, b_r, att):
    f32 = jnp.float32
    npad = _NP - _N
    wcat = jnp.concatenate([W_l, W_r], axis=1)
    bcat = jnp.concatenate([b_l, b_r]).reshape(1, 16)
    mu_t = jnp.pad(Mu, ((0, _KP - _K), (0, 0))).T
    var_t = jnp.pad(Var, ((0, _KP - _K), (0, 0)), constant_values=1.0).T
    w32 = jnp.pad(W, ((0, npad), (0, _KP - _K)), constant_values=-1e30)
    xp = jnp.pad(X, ((0, npad), (0, 0)))
    sp = jnp.pad(S, ((0, npad), (0, 0)))
    att16 = jnp.pad(att, (0, 8))

    stab, dtab, lp, pe, ll = pl.pallas_call(
        _dense_body,
        out_shape=(
            jax.ShapeDtypeStruct((_NP, 40), f32),
            jax.ShapeDtypeStruct((_NP, 8), f32),
            jax.ShapeDtypeStruct((_NP, _KP), f32),
            jax.ShapeDtypeStruct((_NP, _KP), f32),
            jax.ShapeDtypeStruct((1, 1), f32),
        ),
    )(xp, mu_t, var_t, w32, sp, wcat, bcat)

    edge_call = functools.partial(
        pl.kernel,
        out_type=jax.ShapeDtypeStruct((_NC, _NP, _KP), f32),
        mesh=plsc.VectorSubcoreMesh(
            core_axis_name="c", subcore_axis_name="s",
            num_cores=_NC, num_subcores=_NS),
        scratch_types=(
            [pltpu.VMEM_SHARED((_NP, _KP), f32),
             pltpu.VMEM((2, _NCH, _CH), jnp.int32)]
            + [pltpu.VMEM((_CH, 40), f32)] * 4
            + [pltpu.VMEM((_CH, 8), f32)] * 4
            + [pltpu.VMEM((_CH, _KP), f32)] * 4
            + [pltpu.VMEM((16,), f32),
               pltpu.VMEM((_RPT, _KP), f32)]
            + [pltpu.SemaphoreType.DMA] * 8
        ),
        compiler_params=pltpu.CompilerParams(
            needs_layout_passes=False, use_tc_tiling_on_sc=False),
    )(_edge_body)
    ei3 = edge_index.reshape(2, _E // _CH, _CH)
    bparts = edge_call(ei3, stab, dtab, att16, jnp.zeros((_NP, _KP), f32))

    ce = pl.pallas_call(
        _final_body,
        out_shape=jax.ShapeDtypeStruct((1, 1), f32),
    )(bparts, lp)

    return (ll[0, 0], ce[0, 0], pe[:_N, :_K])
